# baseline (speedup 1.0000x reference)
import math

import jax
import jax.numpy as jnp
from jax.experimental import pallas as pl
from jax.experimental.pallas import tpu as pltpu

_F32 = jnp.float32
_BF16 = jnp.bfloat16
_VMEM_LIMIT = 100 * 1024 * 1024


def _compiler_params(**kw):
    cls = getattr(pltpu, "CompilerParams", None) or getattr(pltpu, "TPUCompilerParams")
    return cls(**kw)


_PB = 256


def _score_kernel(featr_ref, w2_ref, featT_ref, w1T_ref, out_ref, f2_ref):
    j = pl.program_id(1)

    @pl.when(j == 0)
    def _():
        f2_ref[...] = jnp.dot(w2_ref[...], featr_ref[...],
                              preferred_element_type=_F32).astype(_BF16)

    x1 = jnp.dot(featT_ref[...], w1T_ref[...],
                 preferred_element_type=_F32).astype(_BF16)
    pieces = []
    for k in range(8):
        for hp in range(4):
            logits = jnp.dot(x1, f2_ref[:, k * 512 + hp * 128:k * 512 + (hp + 1) * 128],
                             preferred_element_type=_F32)
            sp = 1.0 / (1.0 + jnp.exp(-logits))
            pieces.append(sp[:, :64])
            pieces.append(sp[:, 64:])
    cat = jnp.concatenate(pieces, axis=0)
    g = cat.reshape(64, _PB, 64)
    out_ref[...] = jnp.transpose(g, (1, 0, 2))


def _score_volume(featr_aug, featT_aug, w1, b1, w2, b2):
    p = featr_aug.shape[1]
    s = int(math.isqrt(p))
    scale = 1.0 / math.sqrt(128.0)
    w1r = w1.reshape(128, 256)
    w2r = w2.reshape(128, 256)
    w1_aug = jnp.concatenate(
        [w1r * scale, jnp.tile((b1 * scale / 8.0)[:, None], (1, 8))],
        axis=1).astype(_BF16)
    w2_aug = jnp.concatenate(
        [w2r, jnp.tile((b2 / 8.0)[:, None], (1, 8))], axis=1).astype(_BF16)
    w1T_aug = w1_aug.T

    nblk = p // _PB
    out = pl.pallas_call(
        _score_kernel,
        grid=(2, nblk // 2),
        in_specs=[
            pl.BlockSpec((264, p), lambda i, j: (0, 0)),
            pl.BlockSpec((128, 264), lambda i, j: (0, 0)),
            pl.BlockSpec((_PB, 264), lambda i, j: (i * (nblk // 2) + j, 0)),
            pl.BlockSpec((264, 128), lambda i, j: (0, 0)),
        ],
        out_specs=pl.BlockSpec((_PB, 64, 64),
                               lambda i, j: (i * (nblk // 2) + j, 0, 0)),
        out_shape=jax.ShapeDtypeStruct((p, 64, 64), _F32),
        scratch_shapes=[pltpu.VMEM((128, p), _BF16)],
        compiler_params=_compiler_params(
            dimension_semantics=("parallel", "arbitrary"),
            vmem_limit_bytes=_VMEM_LIMIT,
        ),
    )(featr_aug, w2_aug, featT_aug, w1T_aug)
    return out.reshape(s, s, s, s)


def _conv_taps(w, cin_pad=0):
    o, i = w.shape[0], w.shape[1]
    wt = jnp.transpose(w.astype(_BF16).reshape(o * i, 9)).reshape(9, o, i)
    if cin_pad:
        wt = jnp.pad(wt, ((0, 0), (0, 0), (0, cin_pad)))
    return wt


def _dot_tb(a, w_oi):
    return jax.lax.dot_general(a, w_oi, (((1,), (1,)), ((), ())),
                               preferred_element_type=_F32)


def _accum_conv(src_slices, wt_ref):
    acc = None
    for a, t in src_slices:
        contrib = _dot_tb(a, wt_ref[t])
        acc = contrib if acc is None else acc + contrib
    return acc


def _stride1_slices(ref):
    for dy in range(3):
        for dx in range(3):
            a = ref[dy:dy + 32, dx:dx + 32, :].reshape(1024, ref.shape[2])
            yield a.astype(_BF16), dy * 3 + dx


def _write_padded(out_ref, val):
    out_ref[...] = jnp.zeros(out_ref.shape, _F32)
    out_ref[1:33, 1:33, :] = val.reshape(32, 32, out_ref.shape[2])


def _block0_kernel(q00, q01, q10, q11, w1t, w2t, wdw,
                   s1, c1, s2, c2, sd, cd, out_ref, ypad, php):
    php[...] = jnp.zeros(php.shape, _BF16)
    srcs = (q00, q01, q10, q11)
    for py in range(2):
        for px in range(2):
            a, b = 1 - py, 1 - px
            php[py * 2 + px, a:a + 32, b:b + 32, :] = srcs[a * 2 + b][...]

    def stride2_slices():
        for dy in range(3):
            for dx in range(3):
                n = (dy % 2) * 2 + (dx % 2)
                oy, ox = dy // 2, dx // 2
                yield (php[n, oy:oy + 32, ox:ox + 32, :].reshape(1024, 264),
                       dy * 3 + dx)

    y = jnp.maximum(_accum_conv(stride2_slices(), w1t) * s1[...] + c1[...], 0.0)
    _write_padded(ypad, y)
    acc2 = _accum_conv(_stride1_slices(ypad), w2t)
    sc = _dot_tb(q00[...].reshape(1024, 264), wdw[...])
    h = jnp.maximum(acc2 * s2[...] + c2[...] + sc * sd[...] + cd[...], 0.0)
    _write_padded(out_ref, h)


def _block1_kernel(hin, w1t, w2t, s1, c1, s2, c2, out_ref, ypad):
    y = jnp.maximum(_accum_conv(_stride1_slices(hin), w1t) * s1[...] + c1[...], 0.0)
    _write_padded(ypad, y)
    acc2 = _accum_conv(_stride1_slices(ypad), w2t)
    h = jnp.maximum(acc2 * s2[...] + c2[...]
                    + hin[1:33, 1:33, :].reshape(1024, 512), 0.0)
    _write_padded(out_ref, h)


def _block2_kernel(hin, w1t, w2t, s1, c1, s2, c2, fcw, fcb, out_ref, ypad):
    y = jnp.maximum(_accum_conv(_stride1_slices(hin), w1t) * s1[...] + c1[...], 0.0)
    _write_padded(ypad, y)
    acc2 = _accum_conv(_stride1_slices(ypad), w2t)
    h = jnp.maximum(acc2 * s2[...] + c2[...]
                    + hin[1:33, 1:33, :].reshape(1024, 512), 0.0)
    pooled = jnp.sum(h, axis=0, keepdims=True) * (1.0 / 1024.0)
    logits = jnp.dot(pooled, fcw[...], preferred_element_type=_F32) + fcb[...]
    lane = jax.lax.broadcasted_iota(jnp.int32, (1, 128), 1)
    mask = lane < 2
    neg = jnp.where(mask, logits, -1e30)
    m = jnp.max(neg, axis=1, keepdims=True)
    e = jnp.where(mask, jnp.exp(neg - m), 0.0)
    out_ref[...] = e / jnp.sum(e, axis=1, keepdims=True)


def _head(featT_aug, l40c1, l40s1, l40b1, l40c2, l40s2, l40b2,
          l40dw, l40ds, l40db, l41c1, l41s1, l41b1, l41c2, l41s2, l41b2,
          l42c1, l42s1, l42b1, l42c2, l42s2, l42b2, fc_w, fc_b):
    f3 = featT_aug.reshape(64, 64, 264)
    p00 = f3[0::2, 0::2]
    p01 = f3[0::2, 1::2]
    p10 = f3[1::2, 0::2]
    p11 = f3[1::2, 1::2]

    row = lambda v: v.reshape(1, 512)
    params = _compiler_params(vmem_limit_bytes=_VMEM_LIMIT)
    padded = jax.ShapeDtypeStruct((34, 34, 512), _F32)
    ypad_scratch = pltpu.VMEM((34, 34, 512), _F32)

    wdw = jnp.pad(l40dw.reshape(512, 256).astype(_BF16), ((0, 0), (0, 8)))
    h0 = pl.pallas_call(
        _block0_kernel, out_shape=padded,
        scratch_shapes=[ypad_scratch, pltpu.VMEM((4, 33, 33, 264), _BF16)],
        compiler_params=params,
    )(p00, p01, p10, p11, _conv_taps(l40c1, 8), _conv_taps(l40c2), wdw,
      row(l40s1), row(l40b1), row(l40s2), row(l40b2), row(l40ds), row(l40db))

    h1 = pl.pallas_call(
        _block1_kernel, out_shape=padded, scratch_shapes=[ypad_scratch],
        compiler_params=params,
    )(h0, _conv_taps(l41c1), _conv_taps(l41c2), row(l41s1), row(l41b1),
      row(l41s2), row(l41b2))

    fcw = jnp.pad(fc_w.T, ((0, 0), (0, 126)))
    fcb = jnp.pad(fc_b.reshape(1, 2), ((0, 0), (0, 126)))
    lab = pl.pallas_call(
        _block2_kernel, out_shape=jax.ShapeDtypeStruct((1, 128), _F32),
        scratch_shapes=[ypad_scratch], compiler_params=params,
    )(h1, _conv_taps(l42c1), _conv_taps(l42c2), row(l42s1), row(l42b1),
      row(l42s2), row(l42b2), fcw, fcb)
    return lab[0, 0:2]


def kernel(feat, w1, b1, w2, b2, l40c1, l40s1, l40b1, l40c2, l40s2, l40b2,
           l40dw, l40ds, l40db, l41c1, l41s1, l41b1, l41c2, l41s2, l41b2,
           l42c1, l42s1, l42b1, l42c2, l42s2, l42b2, fc_w, fc_b):
    p = feat.shape[2] * feat.shape[3]
    featr = feat[0].reshape(256, p)
    featr_aug = jnp.concatenate([featr, jnp.ones((8, p), _F32)],
                                axis=0).astype(_BF16)
    featT_aug = featr_aug.T
    score_volumn = _score_volume(featr_aug, featT_aug, w1, b1, w2, b2)
    label = _head(featT_aug, l40c1, l40s1, l40b1, l40c2, l40s2, l40b2,
                  l40dw, l40ds, l40db, l41c1, l41s1, l41b1, l41c2, l41s2,
                  l41b2, l42c1, l42s1, l42b1, l42c2, l42s2, l42b2, fc_w, fc_b)
    return (score_volumn, label)


# speedup vs baseline: 1.1761x; 1.1761x over previous
import math

import jax
import jax.numpy as jnp
from jax.experimental import pallas as pl
from jax.experimental.pallas import tpu as pltpu

_F32 = jnp.float32
_BF16 = jnp.bfloat16
_VMEM_LIMIT = 100 * 1024 * 1024


def _compiler_params(**kw):
    cls = getattr(pltpu, "CompilerParams", None) or getattr(pltpu, "TPUCompilerParams")
    return cls(**kw)


_PB = 256


def _score_kernel(featr_ref, w2_ref, featT_ref, w1T_ref, out_ref, f2_ref):
    j = pl.program_id(1)

    @pl.when(j == 0)
    def _():
        f2_ref[...] = jnp.dot(w2_ref[...], featr_ref[...],
                              preferred_element_type=_F32).astype(_BF16)

    x1 = jnp.dot(featT_ref[...], w1T_ref[...],
                 preferred_element_type=_F32).astype(_BF16)
    pieces = []
    for k in range(8):
        for hp in range(4):
            logits = jnp.dot(x1, f2_ref[:, k * 512 + hp * 128:k * 512 + (hp + 1) * 128],
                             preferred_element_type=_F32)
            sp = 1.0 / (1.0 + jnp.exp(-logits))
            pieces.append(sp[:, :64])
            pieces.append(sp[:, 64:])
    cat = jnp.concatenate(pieces, axis=0)
    g = cat.reshape(64, _PB, 64)
    out_ref[...] = jnp.transpose(g, (1, 0, 2))


def _score_volume(featr_aug, featT_aug, w1, b1, w2, b2):
    p = featr_aug.shape[1]
    s = int(math.isqrt(p))
    scale = 1.0 / math.sqrt(128.0)
    w1r = w1.reshape(128, 256)
    w2r = w2.reshape(128, 256)
    w1_aug = jnp.concatenate(
        [w1r * scale, jnp.tile((b1 * scale / 8.0)[:, None], (1, 8))],
        axis=1).astype(_BF16)
    w2_aug = jnp.concatenate(
        [w2r, jnp.tile((b2 / 8.0)[:, None], (1, 8))], axis=1).astype(_BF16)
    w1T_aug = w1_aug.T

    nblk = p // _PB
    out = pl.pallas_call(
        _score_kernel,
        grid=(2, nblk // 2),
        in_specs=[
            pl.BlockSpec((264, p), lambda i, j: (0, 0)),
            pl.BlockSpec((128, 264), lambda i, j: (0, 0)),
            pl.BlockSpec((_PB, 264), lambda i, j: (i * (nblk // 2) + j, 0)),
            pl.BlockSpec((264, 128), lambda i, j: (0, 0)),
        ],
        out_specs=pl.BlockSpec((_PB, 64, 64),
                               lambda i, j: (i * (nblk // 2) + j, 0, 0)),
        out_shape=jax.ShapeDtypeStruct((p, 64, 64), _F32),
        scratch_shapes=[pltpu.VMEM((128, p), _BF16)],
        compiler_params=_compiler_params(
            dimension_semantics=("parallel", "arbitrary"),
            vmem_limit_bytes=_VMEM_LIMIT,
        ),
    )(featr_aug, w2_aug, featT_aug, w1T_aug)
    return out.reshape(s, s, s, s)


def _conv_taps(w, cin_pad=0):
    o, i = w.shape[0], w.shape[1]
    wt = jnp.zeros((9, o, i), _BF16)
    if cin_pad:
        wt = jnp.pad(wt, ((0, 0), (0, 0), (0, cin_pad)))
    return wt


def _dot_tb(a, w_oi):
    return jax.lax.dot_general(a, w_oi, (((1,), (1,)), ((), ())),
                               preferred_element_type=_F32)


def _accum_conv(src_slices, wt_ref):
    acc = None
    for a, t in src_slices:
        contrib = _dot_tb(a, wt_ref[t])
        acc = contrib if acc is None else acc + contrib
    return acc


def _stride1_slices(ref):
    for dy in range(3):
        for dx in range(3):
            a = ref[dy:dy + 32, dx:dx + 32, :].reshape(1024, ref.shape[2])
            yield a.astype(_BF16), dy * 3 + dx


def _write_padded(out_ref, val):
    out_ref[...] = jnp.zeros(out_ref.shape, _F32)
    out_ref[1:33, 1:33, :] = val.reshape(32, 32, out_ref.shape[2])


def _block0_kernel(q00, q01, q10, q11, w1t, w2t, wdw,
                   s1, c1, s2, c2, sd, cd, out_ref, ypad, php):
    php[...] = jnp.zeros(php.shape, _BF16)
    srcs = (q00, q01, q10, q11)
    for py in range(2):
        for px in range(2):
            a, b = 1 - py, 1 - px
            php[py * 2 + px, a:a + 32, b:b + 32, :] = srcs[a * 2 + b][...]

    def stride2_slices():
        for dy in range(3):
            for dx in range(3):
                n = (dy % 2) * 2 + (dx % 2)
                oy, ox = dy // 2, dx // 2
                yield (php[n, oy:oy + 32, ox:ox + 32, :].reshape(1024, 264),
                       dy * 3 + dx)

    y = jnp.maximum(_accum_conv(stride2_slices(), w1t) * s1[...] + c1[...], 0.0)
    _write_padded(ypad, y)
    acc2 = _accum_conv(_stride1_slices(ypad), w2t)
    sc = _dot_tb(q00[...].reshape(1024, 264), wdw[...])
    h = jnp.maximum(acc2 * s2[...] + c2[...] + sc * sd[...] + cd[...], 0.0)
    _write_padded(out_ref, h)


def _block1_kernel(hin, w1t, w2t, s1, c1, s2, c2, out_ref, ypad):
    y = jnp.maximum(_accum_conv(_stride1_slices(hin), w1t) * s1[...] + c1[...], 0.0)
    _write_padded(ypad, y)
    acc2 = _accum_conv(_stride1_slices(ypad), w2t)
    h = jnp.maximum(acc2 * s2[...] + c2[...]
                    + hin[1:33, 1:33, :].reshape(1024, 512), 0.0)
    _write_padded(out_ref, h)


def _block2_kernel(hin, w1t, w2t, s1, c1, s2, c2, fcw, fcb, out_ref, ypad):
    y = jnp.maximum(_accum_conv(_stride1_slices(hin), w1t) * s1[...] + c1[...], 0.0)
    _write_padded(ypad, y)
    acc2 = _accum_conv(_stride1_slices(ypad), w2t)
    h = jnp.maximum(acc2 * s2[...] + c2[...]
                    + hin[1:33, 1:33, :].reshape(1024, 512), 0.0)
    pooled = jnp.sum(h, axis=0, keepdims=True) * (1.0 / 1024.0)
    logits = jnp.dot(pooled, fcw[...], preferred_element_type=_F32) + fcb[...]
    lane = jax.lax.broadcasted_iota(jnp.int32, (1, 128), 1)
    mask = lane < 2
    neg = jnp.where(mask, logits, -1e30)
    m = jnp.max(neg, axis=1, keepdims=True)
    e = jnp.where(mask, jnp.exp(neg - m), 0.0)
    out_ref[...] = e / jnp.sum(e, axis=1, keepdims=True)


def _head(featT_aug, l40c1, l40s1, l40b1, l40c2, l40s2, l40b2,
          l40dw, l40ds, l40db, l41c1, l41s1, l41b1, l41c2, l41s2, l41b2,
          l42c1, l42s1, l42b1, l42c2, l42s2, l42b2, fc_w, fc_b):
    f3 = featT_aug.reshape(64, 64, 264)
    p00 = f3[0::2, 0::2]
    p01 = f3[0::2, 1::2]
    p10 = f3[1::2, 0::2]
    p11 = f3[1::2, 1::2]

    row = lambda v: v.reshape(1, 512)
    params = _compiler_params(vmem_limit_bytes=_VMEM_LIMIT)
    padded = jax.ShapeDtypeStruct((34, 34, 512), _F32)
    ypad_scratch = pltpu.VMEM((34, 34, 512), _F32)

    wdw = jnp.pad(l40dw.reshape(512, 256).astype(_BF16), ((0, 0), (0, 8)))
    h0 = pl.pallas_call(
        _block0_kernel, out_shape=padded,
        scratch_shapes=[ypad_scratch, pltpu.VMEM((4, 33, 33, 264), _BF16)],
        compiler_params=params,
    )(p00, p01, p10, p11, _conv_taps(l40c1, 8), _conv_taps(l40c2), wdw,
      row(l40s1), row(l40b1), row(l40s2), row(l40b2), row(l40ds), row(l40db))

    h1 = pl.pallas_call(
        _block1_kernel, out_shape=padded, scratch_shapes=[ypad_scratch],
        compiler_params=params,
    )(h0, _conv_taps(l41c1), _conv_taps(l41c2), row(l41s1), row(l41b1),
      row(l41s2), row(l41b2))

    fcw = jnp.pad(fc_w.T, ((0, 0), (0, 126)))
    fcb = jnp.pad(fc_b.reshape(1, 2), ((0, 0), (0, 126)))
    lab = pl.pallas_call(
        _block2_kernel, out_shape=jax.ShapeDtypeStruct((1, 128), _F32),
        scratch_shapes=[ypad_scratch], compiler_params=params,
    )(h1, _conv_taps(l42c1), _conv_taps(l42c2), row(l42s1), row(l42b1),
      row(l42s2), row(l42b2), fcw, fcb)
    return lab[0, 0:2]


def kernel(feat, w1, b1, w2, b2, l40c1, l40s1, l40b1, l40c2, l40s2, l40b2,
           l40dw, l40ds, l40db, l41c1, l41s1, l41b1, l41c2, l41s2, l41b2,
           l42c1, l42s1, l42b1, l42c2, l42s2, l42b2, fc_w, fc_b):
    p = feat.shape[2] * feat.shape[3]
    featr = feat[0].reshape(256, p)
    featr_aug = jnp.concatenate([featr, jnp.ones((8, p), _F32)],
                                axis=0).astype(_BF16)
    featT_aug = featr_aug.T
    score_volumn = _score_volume(featr_aug, featT_aug, w1, b1, w2, b2)
    label = _head(featT_aug, l40c1, l40s1, l40b1, l40c2, l40s2, l40b2,
                  l40dw, l40ds, l40db, l41c1, l41s1, l41b1, l41c2, l41s2,
                  l41b2, l42c1, l42s1, l42b1, l42c2, l42s2, l42b2, fc_w, fc_b)
    return (score_volumn, label)


# speedup vs baseline: 1.2040x; 1.0237x over previous
import math

import jax
import jax.numpy as jnp
from jax.experimental import pallas as pl
from jax.experimental.pallas import tpu as pltpu

_F32 = jnp.float32
_BF16 = jnp.bfloat16
_VMEM_LIMIT = 100 * 1024 * 1024


def _compiler_params(**kw):
    cls = getattr(pltpu, "CompilerParams", None) or getattr(pltpu, "TPUCompilerParams")
    return cls(**kw)


_PB = 256


def _score_kernel(featr_ref, w2_ref, featT_ref, w1T_ref, out_ref, f2_ref):
    j = pl.program_id(1)

    @pl.when(j == 0)
    def _():
        f2_ref[...] = jnp.dot(w2_ref[...], featr_ref[...],
                              preferred_element_type=_F32).astype(_BF16)

    x1 = jnp.dot(featT_ref[...], w1T_ref[...],
                 preferred_element_type=_F32).astype(_BF16)
    pieces = []
    for k in range(8):
        for hp in range(4):
            logits = jnp.dot(x1, f2_ref[:, k * 512 + hp * 128:k * 512 + (hp + 1) * 128],
                             preferred_element_type=_F32)
            sp = 1.0 / (1.0 + jnp.exp(-logits))
            pieces.append(sp[:, :64])
            pieces.append(sp[:, 64:])
    cat = jnp.concatenate(pieces, axis=0)
    g = cat.reshape(64, _PB, 64)
    out_ref[...] = jnp.transpose(g, (1, 0, 2))


def _score_volume(featr_aug, featT_aug, w1, b1, w2, b2):
    p = featr_aug.shape[1]
    s = int(math.isqrt(p))
    scale = 1.0 / math.sqrt(128.0)
    w1r = w1.reshape(128, 256)
    w2r = w2.reshape(128, 256)
    w1_aug = jnp.concatenate(
        [w1r * scale, jnp.tile((b1 * scale / 8.0)[:, None], (1, 8))],
        axis=1).astype(_BF16)
    w2_aug = jnp.concatenate(
        [w2r, jnp.tile((b2 / 8.0)[:, None], (1, 8))], axis=1).astype(_BF16)
    w1T_aug = w1_aug.T

    nblk = p // _PB
    out = pl.pallas_call(
        _score_kernel,
        grid=(2, nblk // 2),
        in_specs=[
            pl.BlockSpec((264, p), lambda i, j: (0, 0)),
            pl.BlockSpec((128, 264), lambda i, j: (0, 0)),
            pl.BlockSpec((_PB, 264), lambda i, j: (i * (nblk // 2) + j, 0)),
            pl.BlockSpec((264, 128), lambda i, j: (0, 0)),
        ],
        out_specs=pl.BlockSpec((_PB, 64, 64),
                               lambda i, j: (i * (nblk // 2) + j, 0, 0)),
        out_shape=jax.ShapeDtypeStruct((p, 64, 64), _F32),
        scratch_shapes=[pltpu.VMEM((128, p), _BF16)],
        compiler_params=_compiler_params(
            dimension_semantics=("parallel", "arbitrary"),
            vmem_limit_bytes=_VMEM_LIMIT,
        ),
    )(featr_aug, w2_aug, featT_aug, w1T_aug)
    return out.reshape(s, s, s, s)


def _conv_taps(w, cin_pad=0):
    o, i = w.shape[0], w.shape[1]
    wt = jnp.transpose(w.astype(_BF16).reshape(o * i, 9)).reshape(9, o, i)
    if cin_pad:
        wt = jnp.pad(wt, ((0, 0), (0, 0), (0, cin_pad)))
    return wt


def _dot_tb(a, w_oi):
    return jax.lax.dot_general(a, w_oi, (((1,), (1,)), ((), ())),
                               preferred_element_type=_F32)


def _accum_conv(src_slices, wt_ref):
    acc = None
    for a, t in src_slices:
        contrib = _dot_tb(a, wt_ref[t])
        acc = contrib if acc is None else acc + contrib
    return acc


def _stride1_slices(ref):
    for dy in range(3):
        for dx in range(3):
            a = ref[dy:dy + 32, dx:dx + 32, :].reshape(1024, ref.shape[2])
            yield a.astype(_BF16), dy * 3 + dx


def _write_padded(out_ref, val):
    out_ref[...] = jnp.zeros(out_ref.shape, _F32)
    out_ref[1:33, 1:33, :] = val.reshape(32, 32, out_ref.shape[2])


def _block0_kernel(f3, w1t, w2t, wdw,
                   s1, c1, s2, c2, sd, cd, out_ref, ypad, php):
    v4 = f3[...].reshape(32, 2, 64, 264)
    srcs = {}
    for a in range(2):
        v8 = v4[:, a].reshape(32, 32, 2, 264)
        for b in range(2):
            srcs[a * 2 + b] = v8[:, :, b, :]
    php[...] = jnp.zeros(php.shape, _BF16)
    for py in range(2):
        for px in range(2):
            a, b = 1 - py, 1 - px
            php[py * 2 + px, a:a + 32, b:b + 32, :] = srcs[a * 2 + b]

    def stride2_slices():
        for dy in range(3):
            for dx in range(3):
                n = (dy % 2) * 2 + (dx % 2)
                oy, ox = dy // 2, dx // 2
                yield (php[n, oy:oy + 32, ox:ox + 32, :].reshape(1024, 264),
                       dy * 3 + dx)

    y = jnp.maximum(_accum_conv(stride2_slices(), w1t) * s1[...] + c1[...], 0.0)
    _write_padded(ypad, y)
    acc2 = _accum_conv(_stride1_slices(ypad), w2t)
    sc = _dot_tb(srcs[0].reshape(1024, 264), wdw[...])
    h = jnp.maximum(acc2 * s2[...] + c2[...] + sc * sd[...] + cd[...], 0.0)
    _write_padded(out_ref, h)


def _block1_kernel(hin, w1t, w2t, s1, c1, s2, c2, out_ref, ypad):
    y = jnp.maximum(_accum_conv(_stride1_slices(hin), w1t) * s1[...] + c1[...], 0.0)
    _write_padded(ypad, y)
    acc2 = _accum_conv(_stride1_slices(ypad), w2t)
    h = jnp.maximum(acc2 * s2[...] + c2[...]
                    + hin[1:33, 1:33, :].reshape(1024, 512), 0.0)
    _write_padded(out_ref, h)


def _block2_kernel(hin, w1t, w2t, s1, c1, s2, c2, fcw, fcb, out_ref, ypad):
    y = jnp.maximum(_accum_conv(_stride1_slices(hin), w1t) * s1[...] + c1[...], 0.0)
    _write_padded(ypad, y)
    acc2 = _accum_conv(_stride1_slices(ypad), w2t)
    h = jnp.maximum(acc2 * s2[...] + c2[...]
                    + hin[1:33, 1:33, :].reshape(1024, 512), 0.0)
    pooled = jnp.sum(h, axis=0, keepdims=True) * (1.0 / 1024.0)
    logits = jnp.dot(pooled, fcw[...], preferred_element_type=_F32) + fcb[...]
    lane = jax.lax.broadcasted_iota(jnp.int32, (1, 128), 1)
    mask = lane < 2
    neg = jnp.where(mask, logits, -1e30)
    m = jnp.max(neg, axis=1, keepdims=True)
    e = jnp.where(mask, jnp.exp(neg - m), 0.0)
    out_ref[...] = e / jnp.sum(e, axis=1, keepdims=True)


def _head(featT_aug, l40c1, l40s1, l40b1, l40c2, l40s2, l40b2,
          l40dw, l40ds, l40db, l41c1, l41s1, l41b1, l41c2, l41s2, l41b2,
          l42c1, l42s1, l42b1, l42c2, l42s2, l42b2, fc_w, fc_b):
    f3 = featT_aug.reshape(64, 64, 264)

    row = lambda v: v.reshape(1, 512)
    params = _compiler_params(vmem_limit_bytes=_VMEM_LIMIT)
    padded = jax.ShapeDtypeStruct((34, 34, 512), _F32)
    ypad_scratch = pltpu.VMEM((34, 34, 512), _F32)

    wdw = jnp.pad(l40dw.reshape(512, 256).astype(_BF16), ((0, 0), (0, 8)))
    h0 = pl.pallas_call(
        _block0_kernel, out_shape=padded,
        scratch_shapes=[ypad_scratch, pltpu.VMEM((4, 33, 33, 264), _BF16)],
        compiler_params=params,
    )(f3, _conv_taps(l40c1, 8), _conv_taps(l40c2), wdw,
      row(l40s1), row(l40b1), row(l40s2), row(l40b2), row(l40ds), row(l40db))

    h1 = pl.pallas_call(
        _block1_kernel, out_shape=padded, scratch_shapes=[ypad_scratch],
        compiler_params=params,
    )(h0, _conv_taps(l41c1), _conv_taps(l41c2), row(l41s1), row(l41b1),
      row(l41s2), row(l41b2))

    fcw = jnp.pad(fc_w.T, ((0, 0), (0, 126)))
    fcb = jnp.pad(fc_b.reshape(1, 2), ((0, 0), (0, 126)))
    lab = pl.pallas_call(
        _block2_kernel, out_shape=jax.ShapeDtypeStruct((1, 128), _F32),
        scratch_shapes=[ypad_scratch], compiler_params=params,
    )(h1, _conv_taps(l42c1), _conv_taps(l42c2), row(l42s1), row(l42b1),
      row(l42s2), row(l42b2), fcw, fcb)
    return lab[0, 0:2]


def kernel(feat, w1, b1, w2, b2, l40c1, l40s1, l40b1, l40c2, l40s2, l40b2,
           l40dw, l40ds, l40db, l41c1, l41s1, l41b1, l41c2, l41s2, l41b2,
           l42c1, l42s1, l42b1, l42c2, l42s2, l42b2, fc_w, fc_b):
    p = feat.shape[2] * feat.shape[3]
    featr = feat[0].reshape(256, p)
    featr_aug = jnp.concatenate([featr, jnp.ones((8, p), _F32)],
                                axis=0).astype(_BF16)
    featT_aug = featr_aug.T
    score_volumn = _score_volume(featr_aug, featT_aug, w1, b1, w2, b2)
    label = _head(featT_aug, l40c1, l40s1, l40b1, l40c2, l40s2, l40b2,
                  l40dw, l40ds, l40db, l41c1, l41s1, l41b1, l41c2, l41s2,
                  l41b2, l42c1, l42s1, l42b1, l42c2, l42s2, l42b2, fc_w, fc_b)
    return (score_volumn, label)
